# Initial kernel scaffold; baseline (speedup 1.0000x reference)
import jax
import jax.numpy as jnp
from jax.experimental import pallas as pl


def kernel(edge_index, x, edge_attr, conv_w, conv_b, lstm_wih, lstm_whh, lstm_bih, lstm_bhh, gat_w, att_src, att_dst, edge_w, att_edge, gat_b, mha1_in_w, mha1_in_b, mha1_out_w, mha1_out_b, tf_in_w, tf_in_b, tf_out_w, tf_out_b, tf_lin1_w, tf_lin1_b, tf_lin2_w, tf_lin2_b, ln1_g, ln1_b, ln2_g, ln2_b, out_w, out_b):
    raise NotImplementedError("write your pallas kernel here")


# speedup vs baseline: 14.1430x; 14.1430x over previous
import functools

import jax
import jax.numpy as jnp
from jax import lax
from jax.experimental import pallas as pl
from jax.experimental.pallas import tpu as pltpu
from jax.experimental.pallas import tpu_sc as plsc

N = 10000
E = 320000
F_IN = 8
T = 16
H = 128
HEADS = 4
C = H // HEADS
ED = 4
NQ = 3


_RA = 512
_GA = (N + _RA - 1) // _RA


def _enc_body(x_ref, wc_ref, cb_ref, wih_ref, whh_ref, lb_ref, gatw_ref,
              attsd_ref, temp_ref, xw_ref, a16_ref):
    xb = x_ref[...]
    z = jnp.zeros((_RA, 1, F_IN), jnp.float32)
    xm1 = jnp.concatenate([z, xb[:, :T - 1, :]], axis=1)
    xp1 = jnp.concatenate([xb[:, 1:, :], z], axis=1)
    xcat = jnp.concatenate([xm1, xb, xp1], axis=2).reshape(_RA * T, 3 * F_IN)
    seq = jax.nn.relu(
        jnp.dot(xcat, wc_ref[...], preferred_element_type=jnp.float32)
        + cb_ref[...])
    seq = seq.reshape(_RA, T, H)
    wih = wih_ref[...]
    whh = whh_ref[...]
    lb = lb_ref[...]
    h = jnp.zeros((_RA, H), jnp.float32)
    c = jnp.zeros((_RA, H), jnp.float32)
    for t in range(T):
        g = (jnp.dot(seq[:, t, :], wih, preferred_element_type=jnp.float32)
             + lb
             + jnp.dot(h, whh, preferred_element_type=jnp.float32))
        gi = jax.nn.sigmoid(g[:, 0:H])
        gf = jax.nn.sigmoid(g[:, H:2 * H])
        gg = jnp.tanh(g[:, 2 * H:3 * H])
        go = jax.nn.sigmoid(g[:, 3 * H:4 * H])
        c = gf * c + gi * gg
        h = go * jnp.tanh(c)
    temp_ref[...] = h
    xw = jnp.dot(h, gatw_ref[...], preferred_element_type=jnp.float32)
    xw_ref[...] = xw
    xwr = xw.reshape(_RA, HEADS, C)
    asd = attsd_ref[...]
    a_src = (xwr * asd[None, 0:4, :]).sum(-1)
    a_dst = (xwr * asd[None, 4:8, :]).sum(-1)
    a16_ref[...] = jnp.concatenate(
        [a_src, a_dst, jnp.zeros((_RA, 8), jnp.float32)], axis=1)


def _encoder(x, wc, cb, wih_t, whh_t, lb, gat_w, att_sd):
    return pl.pallas_call(
        _enc_body,
        grid=(_GA,),
        in_specs=[
            pl.BlockSpec((_RA, T, F_IN), lambda i: (i, 0, 0)),
            pl.BlockSpec((3 * F_IN, H), lambda i: (0, 0)),
            pl.BlockSpec((1, H), lambda i: (0, 0)),
            pl.BlockSpec((H, 4 * H), lambda i: (0, 0)),
            pl.BlockSpec((H, 4 * H), lambda i: (0, 0)),
            pl.BlockSpec((1, 4 * H), lambda i: (0, 0)),
            pl.BlockSpec((H, H), lambda i: (0, 0)),
            pl.BlockSpec((8, C), lambda i: (0, 0)),
        ],
        out_specs=[
            pl.BlockSpec((_RA, H), lambda i: (i, 0)),
            pl.BlockSpec((_RA, H), lambda i: (i, 0)),
            pl.BlockSpec((_RA, 16), lambda i: (i, 0)),
        ],
        out_shape=[
            jax.ShapeDtypeStruct((N, H), jnp.float32),
            jax.ShapeDtypeStruct((N, H), jnp.float32),
            jax.ShapeDtypeStruct((N, 16), jnp.float32),
        ],
    )(x, wc, cb, wih_t, whh_t, lb, gat_w, att_sd)


_NC = 2
_NS = 16
_HD = H // _NC
_EB = E // _NS
_BLK = 2000
_NBLK = _EB // _BLK
_CH = 80
_NCHK = _BLK // _CH
_NG = _CH // 16
_NP = 10240
_RW = _NP // _NS
_CPR = 16
_SCR = 32


def _sc_edge(src, dst2, ea, a8, xw_sp, wae):
    mesh = plsc.VectorSubcoreMesh(core_axis_name="c", subcore_axis_name="s")

    @functools.partial(
        pl.kernel,
        out_type=[
            jax.ShapeDtypeStruct((_NC, _NP, _HD), jnp.float32),
            jax.ShapeDtypeStruct((_NP, 16), jnp.float32),
        ],
        mesh=mesh,
        scratch_types=[
            pltpu.VMEM((_BLK,), jnp.int32),
            pltpu.VMEM((_NCHK, _CH), jnp.int32),
            pltpu.VMEM((_BLK, ED), jnp.float32),
            pltpu.VMEM((_CH, _HD), jnp.float32),
            pltpu.VMEM((_CH, _HD), jnp.float32),
            pltpu.VMEM((_CH, 16), jnp.float32),
            pltpu.VMEM((16, HEADS), jnp.float32),
            pltpu.VMEM((16, 16), jnp.float32),
            pltpu.VMEM((_CH, 16), jnp.float32),
            pltpu.VMEM((_CH, 16), jnp.float32),
            pltpu.VMEM((_CPR, _HD), jnp.float32),
            pltpu.VMEM((_SCR, 16), jnp.float32),
            pltpu.VMEM_SHARED((_NP, _HD), jnp.float32),
            pltpu.VMEM_SHARED((_NP, 16), jnp.float32),
            pltpu.SemaphoreType.DMA,
        ],
        compiler_params=pltpu.CompilerParams(
            needs_layout_passes=False, use_tc_tiling_on_sc=False),
    )
    def k(src_hbm, dst2_hbm, ea_hbm, a16_hbm, xw_hbm, wae_hbm,
          num_out, small_out,
          src_v, dst_v, ea_v, rows_v, msg_v, sm_v, ex_v, wae_v,
          asrc_v, adst_v, cpn_v, cps_v, num_sh, small_sh, sem):
        cid = lax.axis_index("c")
        sid = lax.axis_index("s")
        ii = jnp.arange(16, dtype=jnp.int32)
        zf = jnp.zeros((16,), jnp.float32)
        ones = jnp.ones((16,), jnp.float32)

        def zrow(buf, rows, width):
            def zb(i, _):
                for j in range(width // 16):
                    buf[i, pl.ds(j * 16, 16)] = zf
                return 0
            lax.fori_loop(0, rows, zb, 0)
        zrow(cpn_v, _CPR, _HD)
        zrow(cps_v, _SCR, 16)
        zrow(sm_v, _CH, 16)

        def znum(q, _):
            pltpu.sync_copy(cpn_v, num_sh.at[pl.ds(sid * _RW + q * _CPR, _CPR)])
            return 0
        lax.fori_loop(0, _RW // _CPR, znum, 0)

        @pl.when(cid == 0)
        def _():
            def zsmall(q, _):
                pltpu.sync_copy(
                    cps_v, small_sh.at[pl.ds(sid * _RW + q * _SCR, _SCR)])
                return 0
            lax.fori_loop(0, _RW // _SCR, zsmall, 0)

        pltpu.sync_copy(wae_hbm, wae_v)
        wsp = [wae_v[i] for i in range(16)]

        plsc.subcore_barrier()

        def group(k_chunk, g):
            e0 = k_chunk * _CH + g * 16
            gv = e0 + ii
            ea_d = [plsc.load_gather(ea_v, [gv, jnp.full((16,), d, jnp.int32)])
                    for d in range(ED)]
            for h in range(HEADS):
                asrc = plsc.load_gather(
                    asrc_v, [gv, jnp.full((16,), h, jnp.int32)])
                adst = plsc.load_gather(
                    adst_v, [gv, jnp.full((16,), 4 + h, jnp.int32)])
                al = asrc + adst
                for d in range(ED):
                    al = al + ea_d[d] * wsp[d * 4 + h]
                al = jnp.where(al > 0.0, al, 0.2 * al)
                exh = jnp.exp(al)
                plsc.store_scatter(ex_v, [ii, jnp.full((16,), h, jnp.int32)], exh)
                plsc.store_scatter(sm_v, [gv, jnp.full((16,), 5 + h, jnp.int32)], exh)
            for d in range(ED):
                plsc.store_scatter(sm_v, [gv, jnp.full((16,), d, jnp.int32)], ea_d[d])
            plsc.store_scatter(sm_v, [gv, jnp.full((16,), 4, jnp.int32)], ones)

            g0v = g * 16 + ii
            hbase = cid * (_HD // C)

            def fbody(f, _):
                fv = jnp.full((16,), f, jnp.int32)
                hv = jnp.full((16,), hbase + lax.shift_right_logical(f, 5),
                              jnp.int32)
                xcol = plsc.load_gather(rows_v, [g0v, fv])
                excol = plsc.load_gather(ex_v, [ii, hv])
                plsc.store_scatter(msg_v, [g0v, fv], xcol * excol)
                return 0
            lax.fori_loop(0, _HD, fbody, 0)

        def block(b, _):
            m = sid * _NBLK + b
            pltpu.sync_copy(src_hbm.at[m], src_v)
            pltpu.sync_copy(dst2_hbm.at[m], dst_v)
            pltpu.sync_copy(ea_hbm.at[m], ea_v)

            def chunk(k_chunk, _):
                sidx = src_v.at[pl.ds(k_chunk * _CH, _CH)]
                didx = dst_v.at[k_chunk]
                pltpu.async_copy(xw_hbm.at[cid].at[sidx], rows_v, sem).wait()
                pltpu.async_copy(a16_hbm.at[sidx], asrc_v, sem).wait()
                pltpu.async_copy(a16_hbm.at[didx], adst_v, sem).wait()
                for g in range(_NG):
                    group(k_chunk, g)
                pltpu.sync_copy(msg_v, num_sh.at[didx], add=True)

                @pl.when(cid == 0)
                def _():
                    pltpu.sync_copy(sm_v, small_sh.at[didx], add=True)
                return 0
            lax.fori_loop(0, _NCHK, chunk, 0)
            return 0
        lax.fori_loop(0, _NBLK, block, 0)

        plsc.subcore_barrier()

        def cpnum(q, _):
            r0 = sid * _RW + q * _CPR
            pltpu.sync_copy(num_sh.at[pl.ds(r0, _CPR)], cpn_v)
            pltpu.sync_copy(cpn_v, num_out.at[cid, pl.ds(r0, _CPR)])
            return 0
        lax.fori_loop(0, _RW // _CPR, cpnum, 0)

        @pl.when(cid == 0)
        def _():
            def cpsmall(q, _):
                r0 = sid * _RW + q * _SCR
                pltpu.sync_copy(small_sh.at[pl.ds(r0, _SCR)], cps_v)
                pltpu.sync_copy(cps_v, small_out.at[pl.ds(r0, _SCR)])
                return 0
            lax.fori_loop(0, _RW // _SCR, cpsmall, 0)

    return k(src, dst2, ea, a8, xw_sp, wae)


_RB = 512
_GB = (N + _RB - 1) // _RB


def _tail_body(temp_ref, xw_ref, nump_ref, smp_ref, a8_ref, wae_ref,
               gatb_ref, m1_ref, c1_ref, m2_ref, c2_ref, l1w_ref, l1b_ref,
               l2w_ref, l2b_ref, ln1g_ref, ln1b_ref, ln2g_ref, ln2b_ref,
               oww_ref, owb_ref, out_ref):
    sm = smp_ref[...]
    ea_sum = sm[:, 0:4]
    cnt = sm[:, 4:5]
    den_e = sm[:, 5:9]
    la = ea_sum / jnp.maximum(cnt, 1.0)
    deg = cnt + 1.0
    w = wae_ref[...]
    a8v = a8_ref[...]
    al = a8v[:, 0:4] + a8v[:, 4:8]
    for d in range(ED):
        al = al + la[:, d:d + 1] * w[d:d + 1, 0:4]
    al = jnp.where(al > 0.0, al, 0.2 * al)
    exl = jnp.exp(al)
    den = den_e + exl + 1e-16
    xwv = xw_ref[...]
    xwr = xwv.reshape(_RB, HEADS, C)
    num = jnp.concatenate([nump_ref[0], nump_ref[1]], axis=1) \
        .reshape(_RB, HEADS, C) + exl[:, :, None] * xwr
    agg = (num / den[:, :, None]).reshape(_RB, H) / deg
    g = agg + gatb_ref[...]
    graph = jnp.where(g > 0.0, g, jnp.exp(jnp.minimum(g, 0.0)) - 1.0)
    temporal = temp_ref[...]
    fused = temporal + jnp.dot(graph, m1_ref[...],
                               preferred_element_type=jnp.float32) + c1_ref[...]
    a2 = jnp.dot(fused, m2_ref[...],
                 preferred_element_type=jnp.float32) + c2_ref[...]

    def ln(v, gg, bb):
        m = v.mean(axis=1, keepdims=True)
        var = ((v - m) ** 2).mean(axis=1, keepdims=True)
        return (v - m) * jax.lax.rsqrt(var + 1e-5) * gg + bb

    x1 = ln(fused + a2, ln1g_ref[...], ln1b_ref[...])
    ff = jnp.dot(
        jax.nn.relu(jnp.dot(x1, l1w_ref[...],
                            preferred_element_type=jnp.float32) + l1b_ref[...]),
        l2w_ref[...], preferred_element_type=jnp.float32) + l2b_ref[...]
    x2 = ln(x1 + ff, ln2g_ref[...], ln2b_ref[...])
    out_ref[...] = jnp.dot(x2, oww_ref[...],
                           preferred_element_type=jnp.float32) + owb_ref[...]


def _tail(temporal, xw, num_p, small_p, a8, wae8, gat_b, m1, c1, m2, c2,
          l1w, l1b, l2w, l2b, ln1g, ln1b, ln2g, ln2b, oww, owb):
    full = lambda shape: pl.BlockSpec(shape, lambda i: tuple(0 for _ in shape))
    return pl.pallas_call(
        _tail_body,
        grid=(_GB,),
        in_specs=[
            pl.BlockSpec((_RB, H), lambda i: (i, 0)),
            pl.BlockSpec((_RB, H), lambda i: (i, 0)),
            pl.BlockSpec((_NC, _RB, _HD), lambda i: (0, i, 0)),
            pl.BlockSpec((_RB, 16), lambda i: (i, 0)),
            pl.BlockSpec((_RB, 16), lambda i: (i, 0)),
            full((8, 8)),
            full((1, H)),
            full((H, H)), full((1, H)),
            full((H, H)), full((1, H)),
            full((H, 2 * H)), full((1, 2 * H)),
            full((2 * H, H)), full((1, H)),
            full((1, H)), full((1, H)), full((1, H)), full((1, H)),
            full((H, 8)), full((1, 8)),
        ],
        out_specs=pl.BlockSpec((_RB, 8), lambda i: (i, 0)),
        out_shape=jax.ShapeDtypeStruct((N, 8), jnp.float32),
    )(temporal, xw, num_p, small_p, a8, wae8, gat_b, m1, c1, m2, c2,
      l1w, l1b, l2w, l2b, ln1g, ln1b, ln2g, ln2b, oww, owb)


def kernel(edge_index, x, edge_attr, conv_w, conv_b, lstm_wih, lstm_whh,
           lstm_bih, lstm_bhh, gat_w, att_src, att_dst, edge_w, att_edge,
           gat_b, mha1_in_w, mha1_in_b, mha1_out_w, mha1_out_b, tf_in_w,
           tf_in_b, tf_out_w, tf_out_b, tf_lin1_w, tf_lin1_b, tf_lin2_w,
           tf_lin2_b, ln1_g, ln1_b, ln2_g, ln2_b, out_w, out_b):
    f32 = jnp.float32
    src = edge_index[0].astype(jnp.int32).reshape(_NS * _NBLK, _BLK)
    dst = edge_index[1].astype(jnp.int32)
    dst2 = dst.reshape(_NS * _NBLK, _NCHK, _CH)
    ea3 = edge_attr.reshape(_NS * _NBLK, _BLK, ED)

    wc = conv_w.transpose(2, 1, 0).reshape(3 * F_IN, H)
    cb = conv_b.reshape(1, H)
    wih_t = lstm_wih.T
    whh_t = lstm_whh.T
    lb = (lstm_bih + lstm_bhh).reshape(1, 4 * H)
    att_sd = jnp.concatenate([att_src, att_dst], axis=0)
    wae = jnp.einsum('dhc,hc->dh', edge_w.reshape(ED, HEADS, C), att_edge)
    wae_flat = jnp.tile(wae.reshape(16, 1).astype(f32), (1, 16))
    wae8 = jnp.zeros((8, 8), f32).at[0:ED, 0:HEADS].set(wae)
    vw1 = mha1_in_w[2 * H:]
    vb1 = mha1_in_b[2 * H:]
    m1 = (mha1_out_w @ vw1).T
    c1 = (mha1_out_w @ vb1 + mha1_out_b).reshape(1, H)
    vw2 = tf_in_w[2 * H:]
    vb2 = tf_in_b[2 * H:]
    m2 = (tf_out_w @ vw2).T
    c2 = (tf_out_w @ vb2 + tf_out_b).reshape(1, H)
    l1w = tf_lin1_w.T
    l1b = tf_lin1_b.reshape(1, 2 * H)
    l2w = tf_lin2_w.T
    l2b = tf_lin2_b.reshape(1, H)
    oww = jnp.zeros((H, 8), f32).at[:, 0:NQ].set(out_w.T)
    owb = jnp.zeros((1, 8), f32).at[0, 0:NQ].set(out_b)

    temporal, xw, a8 = _encoder(x, wc, cb, wih_t, whh_t, lb, gat_w, att_sd)
    xw_sp = xw.reshape(N, _NC, _HD).transpose(1, 0, 2)
    num_p, small_p = _sc_edge(src, dst2, ea3, a8, xw_sp, wae_flat)
    out8 = _tail(temporal, xw, num_p, small_p, a8, wae8,
                 gat_b.reshape(1, H), m1, c1, m2, c2, l1w, l1b, l2w, l2b,
                 ln1_g.reshape(1, H), ln1_b.reshape(1, H),
                 ln2_g.reshape(1, H), ln2_b.reshape(1, H), oww, owb)
    return out8[:, 0:NQ]
